# baseline (speedup 1.0000x reference)
import functools

import jax
import jax.numpy as jnp
import numpy as np
from jax.experimental import pallas as pl

SCALE_FACTORS = [0.01, 0.03, 0.05, 0.08, 0.12, 0.16, 0.21, 0.27, 0.33, 0.41,
                 0.49, 0.57, 0.67, 0.77, 0.88, 1.0]
N_Q = 16
INPUT_DIM = 512
CB_SIZE = 4096
CB_DIM = 256
N_PHI = 5
KS = 9
T = 1500
BATCH = 2
TICKS = np.linspace(1.0 / (2 * N_PHI), 1.0 - 1.0 / (2 * N_PHI), N_PHI)
BF = jnp.bfloat16
F32 = jnp.float32
HIGHEST = jax.lax.Precision.HIGHEST
CHUNK = 512


def _area_matrix(T_in, S_out):
    A = np.zeros((S_out, T_in), dtype=np.float32)
    for i in range(S_out):
        start = int(np.floor(i * T_in / S_out))
        end = int(np.ceil((i + 1) * T_in / S_out))
        A[i, start:end] = 1.0 / (end - start)
    return A


def _linear_matrix(S_in, T_out):
    M = np.zeros((T_out, S_in), dtype=np.float32)
    if S_in == 1:
        M[:, 0] = 1.0
        return M
    for i in range(T_out):
        x = (i + 0.5) * S_in / T_out - 0.5
        x = min(max(x, 0.0), S_in - 1.0)
        x0 = int(np.floor(x))
        x1 = min(x0 + 1, S_in - 1)
        w = x - x0
        M[i, x0] += 1.0 - w
        M[i, x1] += w
    return M


def _bdot(a, b):
    return jnp.dot(a.astype(BF), b.astype(BF), preferred_element_type=F32)


def _tree_sum(prods):
    while len(prods) > 1:
        nxt = [prods[j] + prods[j + 1] for j in range(0, len(prods) - 1, 2)]
        if len(prods) % 2:
            nxt.append(prods[-1])
        prods = nxt
    return prods[0]


def _encode_kernel(S, res_ref, inwt_ref, inb_ref, a_ref, cbnt_ref, cbsq_ref,
                   cb_ref, ze_ref, zqs_ref):
    z_e = _bdot(res_ref[0], inwt_ref[...]) + inb_ref[...]
    ze_ref[0] = z_e
    enc = jnp.dot(a_ref[...], z_e.astype(BF), preferred_element_type=F32)
    norm = jnp.sqrt(jnp.sum(enc * enc, axis=1, keepdims=True))
    enc_n = enc / jnp.maximum(norm, 1e-12)
    enc_sq = jnp.sum(enc_n * enc_n, axis=1, keepdims=True)
    enc_n_bf = enc_n.astype(BF)
    cbsq = cbsq_ref[...]
    for c0 in range(0, S, CHUNK):
        c1 = min(c0 + CHUNK, S)
        sim = jnp.dot(enc_n_bf[c0:c1], cbnt_ref[...],
                      preferred_element_type=F32)
        dist = enc_sq[c0:c1] - 2.0 * sim + cbsq
        score = -dist
        m = jnp.max(score, axis=1, keepdims=True)
        iota = jax.lax.broadcasted_iota(jnp.int32, score.shape, 1)
        idx = jnp.min(jnp.where(score == m, iota, CB_SIZE), axis=1,
                      keepdims=True)
        onehot = (iota == idx).astype(F32)
        zqs_ref[0, c0:c1, :] = jnp.dot(onehot, cb_ref[...],
                                       precision=HIGHEST,
                                       preferred_element_type=F32)


def _up_kernel(u_ref, zqs_ref, up_ref):
    up_ref[0] = jnp.dot(u_ref[...], zqs_ref[0].astype(BF),
                        preferred_element_type=F32)


def _conv_kernel(up_ref, phiwt_ref, phib_ref, y_ref):
    x_bf = up_ref[0].astype(BF)
    padz = jnp.zeros((4, CB_DIM), dtype=BF)
    xpad = jnp.concatenate([padz, x_bf, padz], axis=0)
    prods = [jnp.dot(xpad[k:k + T, :], phiwt_ref[k],
                     preferred_element_type=F32) for k in range(KS)]
    y_ref[0] = _tree_sum(prods) + phib_ref[...]


def _delta_kernel(ze_ref, up_ref, y_ref, delta_ref, loss_ref):
    mix = up_ref[0] * 0.5 + y_ref[0] * 0.5
    delta = mix - ze_ref[0]
    delta_ref[0] = delta
    commit = jnp.sum(delta * delta) / np.float32(T * CB_DIM)
    lane = jax.lax.broadcasted_iota(jnp.int32, (1, 1, 128), 2)
    loss_ref[...] = jnp.where(lane < 2, commit, 0.0).astype(F32)


def _finish_kernel(ze_ref, delta_ref, outwt_ref, outb_ref, res_ref,
                   zqt_ref, newres_ref, zqtout_ref):
    st = ze_ref[0] + delta_ref[0]
    zqout = _bdot(st, outwt_ref[...]) + outb_ref[...]
    newres_ref[0] = res_ref[0] - zqout
    zqtout_ref[0] = zqt_ref[0] + zqout


def kernel(z, in_w, in_b, out_w, out_b, codebooks, phi_w, phi_b):
    z_t = jnp.transpose(z, (0, 2, 1))
    res_t = z_t
    zqt_t = jnp.zeros((BATCH, T, INPUT_DIM), F32)
    commit_total = jnp.float32(0.0)
    cb_total = jnp.float32(0.0)
    rep2 = lambda b: (0, 0)
    rep3 = lambda b: (0, 0, 0)
    bat3 = lambda b: (b, 0, 0)
    for i in range(N_Q):
        S = int(SCALE_FACTORS[i] * T)
        A_bf = jnp.asarray(_area_matrix(T, S)).astype(BF)
        U_bf = jnp.asarray(_linear_matrix(S, T)).astype(BF)
        inwt = in_w[i].T.astype(BF)
        inb = in_b[i][None, :]
        cb = codebooks[i]
        cb_n = cb / jnp.maximum(jnp.linalg.norm(cb, axis=1, keepdims=True),
                                1e-12)
        cbnt = cb_n.T.astype(BF)
        cbsq = (cb_n ** 2).sum(1)[None, :]
        pi = int(np.argmin(np.abs(TICKS - (i / (N_Q - 1)))))
        phiwt = jnp.transpose(phi_w[pi], (2, 1, 0)).astype(BF)
        phib = phi_b[pi][None, :]
        outwt = out_w[i].T.astype(BF)
        outb = out_b[i][None, :]

        z_e_t, zqs = pl.pallas_call(
            functools.partial(_encode_kernel, S),
            grid=(BATCH,),
            in_specs=[
                pl.BlockSpec((1, T, INPUT_DIM), bat3),
                pl.BlockSpec((INPUT_DIM, CB_DIM), rep2),
                pl.BlockSpec((1, CB_DIM), rep2),
                pl.BlockSpec((S, T), rep2),
                pl.BlockSpec((CB_DIM, CB_SIZE), rep2),
                pl.BlockSpec((1, CB_SIZE), rep2),
                pl.BlockSpec((CB_SIZE, CB_DIM), rep2),
            ],
            out_specs=[
                pl.BlockSpec((1, T, CB_DIM), bat3),
                pl.BlockSpec((1, S, CB_DIM), bat3),
            ],
            out_shape=[
                jax.ShapeDtypeStruct((BATCH, T, CB_DIM), F32),
                jax.ShapeDtypeStruct((BATCH, S, CB_DIM), F32),
            ],
        )(res_t, inwt, inb, A_bf, cbnt, cbsq, cb)

        up = pl.pallas_call(
            _up_kernel, grid=(BATCH,),
            in_specs=[pl.BlockSpec((T, S), rep2),
                      pl.BlockSpec((1, S, CB_DIM), bat3)],
            out_specs=[pl.BlockSpec((1, T, CB_DIM), bat3)],
            out_shape=[jax.ShapeDtypeStruct((BATCH, T, CB_DIM), F32)],
        )(U_bf, zqs)[0]

        y = pl.pallas_call(
            _conv_kernel, grid=(BATCH,),
            in_specs=[pl.BlockSpec((1, T, CB_DIM), bat3),
                      pl.BlockSpec((KS, CB_DIM, CB_DIM), rep3),
                      pl.BlockSpec((1, CB_DIM), rep2)],
            out_specs=[pl.BlockSpec((1, T, CB_DIM), bat3)],
            out_shape=[jax.ShapeDtypeStruct((BATCH, T, CB_DIM), F32)],
        )(up, phiwt, phib)[0]

        delta, loss = pl.pallas_call(
            _delta_kernel, grid=(BATCH,),
            in_specs=[
                pl.BlockSpec((1, T, CB_DIM), bat3),
                pl.BlockSpec((1, T, CB_DIM), bat3),
                pl.BlockSpec((1, T, CB_DIM), bat3),
            ],
            out_specs=[pl.BlockSpec((1, T, CB_DIM), bat3),
                       pl.BlockSpec((1, 1, 128), lambda b: (b, 0, 0))],
            out_shape=[jax.ShapeDtypeStruct((BATCH, T, CB_DIM), F32),
                       jax.ShapeDtypeStruct((BATCH, 1, 128), F32)],
        )(z_e_t, up, y)

        res_t, zqt_t = pl.pallas_call(
            _finish_kernel, grid=(BATCH,),
            in_specs=[
                pl.BlockSpec((1, T, CB_DIM), bat3),
                pl.BlockSpec((1, T, CB_DIM), bat3),
                pl.BlockSpec((CB_DIM, INPUT_DIM), rep2),
                pl.BlockSpec((1, INPUT_DIM), rep2),
                pl.BlockSpec((1, T, INPUT_DIM), bat3),
                pl.BlockSpec((1, T, INPUT_DIM), bat3),
            ],
            out_specs=[pl.BlockSpec((1, T, INPUT_DIM), bat3),
                       pl.BlockSpec((1, T, INPUT_DIM), bat3)],
            out_shape=[jax.ShapeDtypeStruct((BATCH, T, INPUT_DIM), F32),
                       jax.ShapeDtypeStruct((BATCH, T, INPUT_DIM), F32)],
        )(z_e_t, delta, outwt, outb, res_t, zqt_t)
        commit_total = commit_total + (loss[0, 0, 0] + loss[1, 0, 0]) * 0.5
        cb_total = cb_total + (loss[0, 0, 1] + loss[1, 0, 1]) * 0.5

    return (jnp.transpose(zqt_t, (0, 2, 1)),
            jnp.stack([commit_total, cb_total]))
